# Initial kernel scaffold; baseline (speedup 1.0000x reference)
import jax
import jax.numpy as jnp
from jax.experimental import pallas as pl


def kernel(x, edge_index, batch, c1_W1, c1_b1, c1_gamma, c1_beta, c1_W2, c1_b2, c2_W1, c2_b1, c2_gamma, c2_beta, c2_W2, c2_b2, c3_W1, c3_b1, c3_gamma, c3_beta, c3_W2, c3_b2, cls_W, cls_b):
    raise NotImplementedError("write your pallas kernel here")


# speedup vs baseline: 3.5474x; 3.5474x over previous
import functools

import jax
import jax.numpy as jnp
from jax import lax
from jax.experimental import pallas as pl
from jax.experimental.pallas import tpu as pltpu
from jax.experimental.pallas import tpu_sc as plsc

N = 10000
E = 320000
G = 64
N_PAD = 10240
NC = 2
NS = 16
NW = NC * NS
CHUNK = 128
E_PAD = ((E + NW * CHUNK - 1) // (NW * CHUNK)) * NW * CHUNK
CH_COLS = E_PAD // (NS * CHUNK)
CH_EDGES = E_PAD // (NW * CHUNK)
ROWS_PER_TILE = N_PAD // NS


def _sc_agg_body(mode, x_a, x_b, src_hbm, dst_hbm, out_hbm,
                 acc, idx_v, dst_v, rows_v, sem):
    c = lax.axis_index("c")
    s = lax.axis_index("s")
    r0 = s * ROWS_PER_TILE

    @pl.when(c == 0)
    def _():
        pltpu.sync_copy(x_a.at[pl.ds(r0, ROWS_PER_TILE)],
                        acc.at[pl.ds(r0, ROWS_PER_TILE)])

    @pl.when(c == 1)
    def _():
        pltpu.sync_copy(x_b.at[pl.ds(r0, ROWS_PER_TILE)],
                        acc.at[pl.ds(r0, ROWS_PER_TILE)])

    plsc.subcore_barrier()

    if mode == "cols":
        e0 = s * (CH_COLS * CHUNK)
        n_chunks = CH_COLS
    else:
        e0 = (c * NS + s) * (CH_EDGES * CHUNK)
        n_chunks = CH_EDGES

    def chunk(j, carry):
        off = e0 + j * CHUNK
        pltpu.sync_copy(src_hbm.at[pl.ds(off, CHUNK)], idx_v)
        pltpu.sync_copy(dst_hbm.at[pl.ds(off, CHUNK)], dst_v)
        if mode == "cols":
            @pl.when(c == 0)
            def _():
                pltpu.async_copy(x_a.at[idx_v], rows_v, sem).wait()

            @pl.when(c == 1)
            def _():
                pltpu.async_copy(x_b.at[idx_v], rows_v, sem).wait()
        else:
            pltpu.async_copy(x_a.at[idx_v], rows_v, sem).wait()
        pltpu.sync_copy(rows_v, acc.at[dst_v], add=True)
        return carry

    lax.fori_loop(0, n_chunks, chunk, 0)
    plsc.subcore_barrier()

    pltpu.sync_copy(acc.at[pl.ds(r0, ROWS_PER_TILE)],
                    out_hbm.at[c, pl.ds(r0, ROWS_PER_TILE)])


def _sc_agg(mode, x_a, x_b, src, dst):
    mesh = plsc.VectorSubcoreMesh(core_axis_name="c", subcore_axis_name="s")
    fn = pl.kernel(
        functools.partial(_sc_agg_body, mode),
        out_type=jax.ShapeDtypeStruct((2, N_PAD, 128), jnp.float32),
        mesh=mesh,
        scratch_types=[
            pltpu.VMEM_SHARED((N_PAD, 128), jnp.float32),
            pltpu.VMEM((CHUNK,), jnp.int32),
            pltpu.VMEM((CHUNK,), jnp.int32),
            pltpu.VMEM((CHUNK, 128), jnp.float32),
            pltpu.SemaphoreType.DMA,
        ],
    )
    return fn(x_a, x_b, src, dst)


def _tc_mlp_body(mode, h_ref, W1_ref, b1_ref, gamma_ref, beta_ref, W2_ref,
                 b2_ref, out_lo_ref, out_hi_ref):
    if mode == "cols":
        h1 = (jnp.dot(h_ref[0], W1_ref[:128], preferred_element_type=jnp.float32)
              + jnp.dot(h_ref[1], W1_ref[128:], preferred_element_type=jnp.float32)
              + b1_ref[...])
    else:
        h1 = (jnp.dot(h_ref[0] + h_ref[1], W1_ref[...],
                      preferred_element_type=jnp.float32) + b1_ref[...])
    v = h1[:N]
    mean = jnp.sum(v, axis=0) / N
    cent = v - mean
    var = jnp.sum(cent * cent, axis=0) / N
    hbn = (h1 - mean) * lax.rsqrt(var + 1e-5) * gamma_ref[...] + beta_ref[...]
    h2 = jnp.maximum(hbn, 0.0)
    h3 = jnp.dot(h2, W2_ref[...], preferred_element_type=jnp.float32) + b2_ref[...]
    h3 = jnp.maximum(h3, 0.0)
    out_lo_ref[...] = h3[:, :128]
    out_hi_ref[...] = h3[:, 128:]


def _tc_mlp(mode, h2x, W1, b1, gamma, beta, W2, b2):
    return pl.pallas_call(
        functools.partial(_tc_mlp_body, mode),
        out_shape=[jax.ShapeDtypeStruct((N_PAD, 128), jnp.float32),
                   jax.ShapeDtypeStruct((N_PAD, 128), jnp.float32)],
    )(h2x, W1, b1, gamma, beta, W2, b2)


def _tc_head_body(b2d_ref, x1l, x1h, x2l, x2h, x3l, x3h, W_ref, b_ref, out_ref):
    gid = lax.broadcasted_iota(jnp.int32, (N, G), 1)
    P = jnp.where(b2d_ref[...] == gid, 1.0, 0.0)
    dn = (((0,), (0,)), ((), ()))
    acc = b_ref[...]
    for i, xr in enumerate((x1l, x1h, x2l, x2h, x3l, x3h)):
        p = lax.dot_general(P, xr[:N], dn, preferred_element_type=jnp.float32)
        acc = acc + jnp.dot(p, W_ref[i * 128:(i + 1) * 128],
                            preferred_element_type=jnp.float32)
    out_ref[...] = acc


def _tc_head(b2d, x1l, x1h, x2l, x2h, x3l, x3h, W_pad, b_pad):
    return pl.pallas_call(
        _tc_head_body,
        out_shape=jax.ShapeDtypeStruct((G, 128), jnp.float32),
    )(b2d, x1l, x1h, x2l, x2h, x3l, x3h, W_pad, b_pad)


def kernel(x, edge_index, batch,
           c1_W1, c1_b1, c1_gamma, c1_beta, c1_W2, c1_b2,
           c2_W1, c2_b1, c2_gamma, c2_beta, c2_W2, c2_b2,
           c3_W1, c3_b1, c3_gamma, c3_beta, c3_W2, c3_b2,
           cls_W, cls_b):
    src = jnp.concatenate([edge_index[0], jnp.zeros((E_PAD - E,), jnp.int32)])
    dst = jnp.concatenate([edge_index[1], jnp.full((E_PAD - E,), N, jnp.int32)])

    xp = jnp.pad(x, ((0, N_PAD - N), (0, 0)))
    zeros = jnp.zeros((N_PAD, 128), jnp.float32)

    a1 = _sc_agg("edges", xp, zeros, src, dst)
    x1l, x1h = _tc_mlp("edges", a1, c1_W1, c1_b1, c1_gamma, c1_beta, c1_W2, c1_b2)

    a2 = _sc_agg("cols", x1l, x1h, src, dst)
    x2l, x2h = _tc_mlp("cols", a2, c2_W1, c2_b1, c2_gamma, c2_beta, c2_W2, c2_b2)

    a3 = _sc_agg("cols", x2l, x2h, src, dst)
    x3l, x3h = _tc_mlp("cols", a3, c3_W1, c3_b1, c3_gamma, c3_beta, c3_W2, c3_b2)

    b2d = jnp.broadcast_to(batch[:, None], (N, G))
    W_pad = jnp.pad(cls_W, ((0, 0), (0, 118)))
    b_pad = jnp.pad(cls_b, ((0, 118)))
    logits = _tc_head(b2d, x1l, x1h, x2l, x2h, x3l, x3h, W_pad, b_pad)
    return logits[:, :10]


# speedup vs baseline: 9.0617x; 2.5545x over previous
import functools

import jax
import jax.numpy as jnp
from jax import lax
from jax.experimental import pallas as pl
from jax.experimental.pallas import tpu as pltpu
from jax.experimental.pallas import tpu_sc as plsc

N = 10000
E = 320000
G = 64
N_PAD = 10240
NC = 2
NS = 16
NW = NC * NS
CHUNK = 128
CH_EDGES = 80
E_PAD = NW * CH_EDGES * CHUNK
CH_COLS = E_PAD // (NS * CHUNK)
N_CHUNKS = E_PAD // CHUNK
ROWS_PER_TILE = N_PAD // NS


def _sc_agg_body(mode, x_a, x_b, epk, out_hbm,
                 acc, ebuf, rows, se0, se1, sg0, sg1):
    c = lax.axis_index("c")
    s = lax.axis_index("s")
    r0 = s * ROWS_PER_TILE
    se = (se0, se1)
    sg = (sg0, sg1)

    @pl.when(c == 0)
    def _():
        pltpu.sync_copy(x_a.at[pl.ds(r0, ROWS_PER_TILE)],
                        acc.at[pl.ds(r0, ROWS_PER_TILE)])

    @pl.when(c == 1)
    def _():
        pltpu.sync_copy(x_b.at[pl.ds(r0, ROWS_PER_TILE)],
                        acc.at[pl.ds(r0, ROWS_PER_TILE)])

    plsc.subcore_barrier()

    if mode == "cols":
        j0 = s * CH_COLS
        n_ch = CH_COLS
    else:
        j0 = (c * NS + s) * CH_EDGES
        n_ch = CH_EDGES

    def start_gather(b):
        if mode == "cols":
            @pl.when(c == 0)
            def _():
                pltpu.async_copy(x_a.at[ebuf.at[b, 0]], rows.at[b], sg[b])

            @pl.when(c == 1)
            def _():
                pltpu.async_copy(x_b.at[ebuf.at[b, 0]], rows.at[b], sg[b])
        else:
            pltpu.async_copy(x_a.at[ebuf.at[b, 0]], rows.at[b], sg[b])

    def drain_e(b):
        pltpu.make_async_copy(epk.at[0], ebuf.at[b], se[b]).wait()

    def drain_g(b):
        pltpu.make_async_copy(x_a.at[pl.ds(0, CHUNK)], rows.at[b],
                              sg[b]).wait()

    pltpu.async_copy(epk.at[j0], ebuf.at[0], se[0])
    pltpu.async_copy(epk.at[j0 + 1], ebuf.at[1], se[1])
    drain_e(0)
    start_gather(0)

    def pair(g_i, carry):
        for b in (0, 1):
            t = g_i * 2 + b
            nb = 1 - b
            drain_e(nb)
            start_gather(nb)
            drain_g(b)
            pltpu.sync_copy(rows.at[b], acc.at[ebuf.at[b, 1]], add=True)
            nxt = jnp.minimum(t + 2, n_ch - 1)
            pltpu.async_copy(epk.at[j0 + nxt], ebuf.at[b], se[b])
        return carry

    lax.fori_loop(0, n_ch // 2, pair, 0)
    drain_e(1)
    drain_g(0)
    plsc.subcore_barrier()

    pltpu.sync_copy(acc.at[pl.ds(r0, ROWS_PER_TILE)],
                    out_hbm.at[c, pl.ds(r0, ROWS_PER_TILE)])


def _sc_agg(mode, x_a, x_b, epk):
    mesh = plsc.VectorSubcoreMesh(core_axis_name="c", subcore_axis_name="s")
    fn = pl.kernel(
        functools.partial(_sc_agg_body, mode),
        out_type=jax.ShapeDtypeStruct((2, N_PAD, 128), jnp.float32),
        mesh=mesh,
        scratch_types=[
            pltpu.VMEM_SHARED((N_PAD, 128), jnp.float32),
            pltpu.VMEM((2, 2, CHUNK), jnp.int32),
            pltpu.VMEM((2, CHUNK, 128), jnp.float32),
            pltpu.SemaphoreType.DMA,
            pltpu.SemaphoreType.DMA,
            pltpu.SemaphoreType.DMA,
            pltpu.SemaphoreType.DMA,
        ],
    )
    return fn(x_a, x_b, epk)


def _tc_mlp_body(mode, h_ref, W1_ref, b1_ref, gamma_ref, beta_ref, W2_ref,
                 b2_ref, out_lo_ref, out_hi_ref):
    if mode == "cols":
        h1 = (jnp.dot(h_ref[0], W1_ref[:128], preferred_element_type=jnp.float32)
              + jnp.dot(h_ref[1], W1_ref[128:], preferred_element_type=jnp.float32)
              + b1_ref[...])
    else:
        h1 = (jnp.dot(h_ref[0] + h_ref[1], W1_ref[...],
                      preferred_element_type=jnp.float32) + b1_ref[...])
    v = h1[:N]
    mean = jnp.sum(v, axis=0) / N
    cent = v - mean
    var = jnp.sum(cent * cent, axis=0) / N
    hbn = (h1 - mean) * lax.rsqrt(var + 1e-5) * gamma_ref[...] + beta_ref[...]
    h2 = jnp.maximum(hbn, 0.0)
    h3 = jnp.dot(h2, W2_ref[...], preferred_element_type=jnp.float32) + b2_ref[...]
    h3 = jnp.maximum(h3, 0.0)
    out_lo_ref[...] = h3[:, :128]
    out_hi_ref[...] = h3[:, 128:]


def _tc_mlp(mode, h2x, W1, b1, gamma, beta, W2, b2):
    return pl.pallas_call(
        functools.partial(_tc_mlp_body, mode),
        out_shape=[jax.ShapeDtypeStruct((N_PAD, 128), jnp.float32),
                   jax.ShapeDtypeStruct((N_PAD, 128), jnp.float32)],
    )(h2x, W1, b1, gamma, beta, W2, b2)


def _tc_head_body(b2d_ref, x1l, x1h, x2l, x2h, x3l, x3h, W_ref, b_ref, out_ref):
    gid = lax.broadcasted_iota(jnp.int32, (N, G), 1)
    P = jnp.where(b2d_ref[...] == gid, 1.0, 0.0)
    dn = (((0,), (0,)), ((), ()))
    acc = b_ref[...]
    for i, xr in enumerate((x1l, x1h, x2l, x2h, x3l, x3h)):
        p = lax.dot_general(P, xr[:N], dn, preferred_element_type=jnp.float32)
        acc = acc + jnp.dot(p, W_ref[i * 128:(i + 1) * 128],
                            preferred_element_type=jnp.float32)
    out_ref[...] = acc


def _tc_head(b2d, x1l, x1h, x2l, x2h, x3l, x3h, W_pad, b_pad):
    return pl.pallas_call(
        _tc_head_body,
        out_shape=jax.ShapeDtypeStruct((G, 128), jnp.float32),
    )(b2d, x1l, x1h, x2l, x2h, x3l, x3h, W_pad, b_pad)


def kernel(x, edge_index, batch,
           c1_W1, c1_b1, c1_gamma, c1_beta, c1_W2, c1_b2,
           c2_W1, c2_b1, c2_gamma, c2_beta, c2_W2, c2_b2,
           c3_W1, c3_b1, c3_gamma, c3_beta, c3_W2, c3_b2,
           cls_W, cls_b):
    pad_n = E_PAD - E
    pad_i = jnp.arange(pad_n, dtype=jnp.int32)
    src = jnp.concatenate([edge_index[0], pad_i % N])
    dst = jnp.concatenate([edge_index[1], N + pad_i % (N_PAD - N)])
    epk = jnp.stack([src.reshape(-1, CHUNK), dst.reshape(-1, CHUNK)], axis=1)

    xp = jnp.pad(x, ((0, N_PAD - N), (0, 0)))
    zeros = jnp.zeros((N_PAD, 128), jnp.float32)

    a1 = _sc_agg("edges", xp, zeros, epk)
    x1l, x1h = _tc_mlp("edges", a1, c1_W1, c1_b1, c1_gamma, c1_beta, c1_W2, c1_b2)

    a2 = _sc_agg("cols", x1l, x1h, epk)
    x2l, x2h = _tc_mlp("cols", a2, c2_W1, c2_b1, c2_gamma, c2_beta, c2_W2, c2_b2)

    a3 = _sc_agg("cols", x2l, x2h, epk)
    x3l, x3h = _tc_mlp("cols", a3, c3_W1, c3_b1, c3_gamma, c3_beta, c3_W2, c3_b2)

    b2d = jnp.broadcast_to(batch[:, None], (N, G))
    W_pad = jnp.pad(cls_W, ((0, 0), (0, 118)))
    b_pad = jnp.pad(cls_b, ((0, 118)))
    logits = _tc_head(b2d, x1l, x1h, x2l, x2h, x3l, x3h, W_pad, b_pad)
    return logits[:, :10]


# speedup vs baseline: 10.4369x; 1.1518x over previous
import functools

import jax
import jax.numpy as jnp
from jax import lax
from jax.experimental import pallas as pl
from jax.experimental.pallas import tpu as pltpu
from jax.experimental.pallas import tpu_sc as plsc

N = 10000
E = 320000
G = 64
N_PAD = 10240
NC = 2
NS = 16
NW = NC * NS
CHUNK = 128
CH_EDGES = 80
E_PAD = NW * CH_EDGES * CHUNK
CH_COLS = E_PAD // (NS * CHUNK)
N_CHUNKS = E_PAD // CHUNK
ROWS_PER_TILE = N_PAD // NS


def _sc_agg_body(mode, x_a, x_b, epk, out_hbm,
                 acc, ebuf, rows, se0, se1, se2, se3, sg0, sg1, ss0, ss1):
    c = lax.axis_index("c")
    s = lax.axis_index("s")
    r0 = s * ROWS_PER_TILE
    se = (se0, se1, se2, se3)
    sg = (sg0, sg1)
    ss = (ss0, ss1)

    @pl.when(c == 0)
    def _():
        pltpu.sync_copy(x_a.at[pl.ds(r0, ROWS_PER_TILE)],
                        acc.at[pl.ds(r0, ROWS_PER_TILE)])

    @pl.when(c == 1)
    def _():
        pltpu.sync_copy(x_b.at[pl.ds(r0, ROWS_PER_TILE)],
                        acc.at[pl.ds(r0, ROWS_PER_TILE)])

    plsc.subcore_barrier()

    if mode == "cols":
        j0 = s * CH_COLS
        n_ch = CH_COLS
    else:
        j0 = (c * NS + s) * CH_EDGES
        n_ch = CH_EDGES

    def start_gather(b, e):
        if mode == "cols":
            @pl.when(c == 0)
            def _():
                pltpu.async_copy(x_a.at[ebuf.at[e, 0]], rows.at[b], sg[b])

            @pl.when(c == 1)
            def _():
                pltpu.async_copy(x_b.at[ebuf.at[e, 0]], rows.at[b], sg[b])
        else:
            pltpu.async_copy(x_a.at[ebuf.at[e, 0]], rows.at[b], sg[b])

    def start_scatter(b, e):
        pltpu.async_copy(rows.at[b], acc.at[ebuf.at[e, 1]], ss[b], add=True)

    def prefetch(j, e):
        pltpu.async_copy(epk.at[j0 + j], ebuf.at[e], se[e])

    def drain_e(e):
        pltpu.make_async_copy(epk.at[0], ebuf.at[e], se[e]).wait()

    def drain_g(b):
        pltpu.make_async_copy(x_a.at[pl.ds(0, CHUNK)], rows.at[b],
                              sg[b]).wait()

    def drain_s(b):
        pltpu.make_async_copy(x_a.at[pl.ds(0, CHUNK)], rows.at[b],
                              ss[b]).wait()

    def step(t, b, e, first=False):
        nb = 1 - b
        en = (e + 1) % 4
        ep = (e + 3) % 4
        if not first:
            drain_s(nb)
        drain_e(en)
        start_gather(nb, en)
        prefetch(jnp.minimum(t + 3, n_ch - 1), ep)
        drain_g(b)
        start_scatter(b, e)

    for e in range(3):
        prefetch(e, e)
    drain_e(0)
    start_gather(0, 0)
    for t in range(4):
        step(t, t % 2, t % 4, first=(t == 0))

    def quad(q, carry):
        t0 = q * 4
        for i in range(4):
            step(t0 + i, i % 2, i, first=False)
        return carry

    lax.fori_loop(1, n_ch // 4, quad, 0)
    drain_s(1)
    drain_g(0)
    drain_e(1)
    drain_e(2)
    plsc.subcore_barrier()

    pltpu.sync_copy(acc.at[pl.ds(r0, ROWS_PER_TILE)],
                    out_hbm.at[c, pl.ds(r0, ROWS_PER_TILE)])


def _sc_agg(mode, x_a, x_b, epk):
    mesh = plsc.VectorSubcoreMesh(core_axis_name="c", subcore_axis_name="s")
    fn = pl.kernel(
        functools.partial(_sc_agg_body, mode),
        out_type=jax.ShapeDtypeStruct((2, N_PAD, 128), jnp.float32),
        mesh=mesh,
        scratch_types=[
            pltpu.VMEM_SHARED((N_PAD, 128), jnp.float32),
            pltpu.VMEM((4, 2, CHUNK), jnp.int32),
            pltpu.VMEM((2, CHUNK, 128), jnp.float32),
            pltpu.SemaphoreType.DMA,
            pltpu.SemaphoreType.DMA,
            pltpu.SemaphoreType.DMA,
            pltpu.SemaphoreType.DMA,
            pltpu.SemaphoreType.DMA,
            pltpu.SemaphoreType.DMA,
            pltpu.SemaphoreType.DMA,
            pltpu.SemaphoreType.DMA,
        ],
    )
    return fn(x_a, x_b, epk)


def _tc_mlp_body(mode, last, h_ref, W1_ref, b1_ref, gamma_ref, beta_ref,
                 W2_ref, b2_ref, b2d_ref, Wc_ref, part_ref, *out_refs):
    if mode == "cols":
        h1 = (jnp.dot(h_ref[0], W1_ref[:128], preferred_element_type=jnp.float32)
              + jnp.dot(h_ref[1], W1_ref[128:], preferred_element_type=jnp.float32)
              + b1_ref[...])
    else:
        h1 = (jnp.dot(h_ref[0] + h_ref[1], W1_ref[...],
                      preferred_element_type=jnp.float32) + b1_ref[...])
    v = h1[:N]
    mean = jnp.sum(v, axis=0) / N
    cent = v - mean
    var = jnp.sum(cent * cent, axis=0) / N
    hbn = (h1 - mean) * lax.rsqrt(var + 1e-5) * gamma_ref[...] + beta_ref[...]
    h2 = jnp.maximum(hbn, 0.0)
    h3 = jnp.dot(h2, W2_ref[...], preferred_element_type=jnp.float32) + b2_ref[...]
    h3 = jnp.maximum(h3, 0.0)
    gid = lax.broadcasted_iota(jnp.int32, (N, G), 1)
    P = jnp.where(b2d_ref[...] == gid, 1.0, 0.0)
    p = lax.dot_general(P, h3[:N], (((0,), (0,)), ((), ())),
                        preferred_element_type=jnp.float32)
    part = part_ref[...] + jnp.dot(p, Wc_ref[...],
                                   preferred_element_type=jnp.float32)
    if last:
        out_refs[0][...] = part
    else:
        out_refs[0][...] = h3[:, :128]
        out_refs[1][...] = h3[:, 128:]
        out_refs[2][...] = part


def _tc_mlp(mode, last, h2x, W1, b1, gamma, beta, W2, b2, b2d, Wc, part):
    if last:
        shapes = [jax.ShapeDtypeStruct((G, 128), jnp.float32)]
    else:
        shapes = [jax.ShapeDtypeStruct((N_PAD, 128), jnp.float32),
                  jax.ShapeDtypeStruct((N_PAD, 128), jnp.float32),
                  jax.ShapeDtypeStruct((G, 128), jnp.float32)]
    return pl.pallas_call(
        functools.partial(_tc_mlp_body, mode, last),
        out_shape=shapes,
    )(h2x, W1, b1, gamma, beta, W2, b2, b2d, Wc, part)


def kernel(x, edge_index, batch,
           c1_W1, c1_b1, c1_gamma, c1_beta, c1_W2, c1_b2,
           c2_W1, c2_b1, c2_gamma, c2_beta, c2_W2, c2_b2,
           c3_W1, c3_b1, c3_gamma, c3_beta, c3_W2, c3_b2,
           cls_W, cls_b):
    pad_n = E_PAD - E
    pad_i = jnp.arange(pad_n, dtype=jnp.int32)
    src = jnp.concatenate([edge_index[0], pad_i % N])
    dst = jnp.concatenate([edge_index[1], N + pad_i % (N_PAD - N)])
    epk = jnp.stack([src.reshape(-1, CHUNK), dst.reshape(-1, CHUNK)], axis=1)

    xp = jnp.pad(x, ((0, N_PAD - N), (0, 0)))
    zeros = jnp.zeros((N_PAD, 128), jnp.float32)
    b2d = jnp.broadcast_to(batch[:, None], (N, G))
    W_pad = jnp.pad(cls_W, ((0, 0), (0, 118)))
    part0 = jnp.broadcast_to(jnp.pad(cls_b, ((0, 118)))[None, :], (G, 128))

    a1 = _sc_agg("edges", xp, zeros, epk)
    x1l, x1h, part1 = _tc_mlp("edges", False, a1, c1_W1, c1_b1, c1_gamma,
                              c1_beta, c1_W2, c1_b2, b2d, W_pad[:256], part0)

    a2 = _sc_agg("cols", x1l, x1h, epk)
    x2l, x2h, part2 = _tc_mlp("cols", False, a2, c2_W1, c2_b1, c2_gamma,
                              c2_beta, c2_W2, c2_b2, b2d, W_pad[256:512], part1)

    a3 = _sc_agg("cols", x2l, x2h, epk)
    (logits,) = _tc_mlp("cols", True, a3, c3_W1, c3_b1, c3_gamma,
                        c3_beta, c3_W2, c3_b2, b2d, W_pad[512:], part2)
    return logits[:, :10]
